# baseline (device time: 653621 ns/iter reference)
import jax
import jax.numpy as jnp
from jax import lax
from jax.experimental import pallas as pl
from jax.experimental.pallas import tpu as pltpu

N_DEV = 4
NC = 1024


def kernel(x, w_mat):
    x = x.astype(jnp.bfloat16)
    m, k_shard = x.shape
    _, n = w_mat.shape
    G = n // NC
    W = NC // 2
    Wc = W // 2
    H = m // 2
    Q = m // 4

    def body(x_hbm, w_ref, out_ref, x_vmem, wb, part, s1r, har, s2r,
             ssem, rsem, xsem):
        my = lax.axis_index("i")
        j = pl.program_id(0)
        pb = j % 2
        pbm = 1 - pb

        def params(c):
            if c == 0:
                p1, p2 = my ^ 1, 3 - my
                h = (my & 1) ^ (my >> 1)
                q = my >> 1
            else:
                p1, p2 = 3 - my, my ^ 1
                h = my >> 1
                q = my & 1
            return p1, p2, h, q

        P = [params(0), params(1)]

        def copy(src, dst, c, k, dev):
            return pltpu.make_async_remote_copy(
                src_ref=src, dst_ref=dst,
                send_sem=ssem.at[c, k], recv_sem=rsem.at[c, k],
                device_id=(dev,), device_id_type=pl.DeviceIdType.MESH,
            )

        def myq_rows(c):
            _, _, h, q = P[c]
            return pl.ds((2 * h + q) * Q, Q)

        def otherq_rows(c):
            _, _, h, q = P[c]
            return pl.ds((2 * h + (1 - q)) * Q, Q)

        cols = [pl.ds(c * W, W) for c in (0, 1)]
        cols_a = [pl.ds(c * W, Wc) for c in (0, 1)]
        cols_b = [pl.ds(c * W + Wc, Wc) for c in (0, 1)]

        r1 = [copy(part.at[c, pl.ds((1 - P[c][2]) * H, H)],
                   s1r.at[pb, c], c, pb, P[c][0]) for c in (0, 1)]
        r2_new = [copy(har.at[c, pl.ds((1 - P[c][3]) * Q, Q)],
                       s2r.at[pb, c], c, 2 + pb, P[c][1]) for c in (0, 1)]
        r2_old = [copy(har.at[c, pl.ds((1 - P[c][3]) * Q, Q)],
                       s2r.at[pbm, c], c, 2 + pbm, P[c][1]) for c in (0, 1)]
        r3a = [copy(out_ref.at[myq_rows(c), cols_a[c]],
                    out_ref.at[myq_rows(c), cols_a[c]],
                    c, 4, P[c][1]) for c in (0, 1)]
        r3b = [copy(out_ref.at[myq_rows(c), cols_b[c]],
                    out_ref.at[myq_rows(c), cols_b[c]],
                    c, 5, P[c][1]) for c in (0, 1)]
        r4a = [copy(out_ref.at[myq_rows(c), cols[c]],
                    out_ref.at[myq_rows(c), cols[c]],
                    c, 6, P[c][0]) for c in (0, 1)]
        r4ba = [copy(out_ref.at[otherq_rows(c), cols_a[c]],
                     out_ref.at[otherq_rows(c), cols_a[c]],
                     c, 7, P[c][0]) for c in (0, 1)]
        r4bb = [copy(out_ref.at[otherq_rows(c), cols_b[c]],
                     out_ref.at[otherq_rows(c), cols_b[c]],
                     c, 8, P[c][0]) for c in (0, 1)]

        barrier_sem = pltpu.get_barrier_semaphore()

        @pl.when(j == 0)
        def _entry():
            xcp = pltpu.make_async_copy(x_hbm, x_vmem, xsem)
            xcp.start()
            for nbr in (my ^ 1, 3 - my):
                pl.semaphore_signal(
                    barrier_sem, inc=1,
                    device_id=(nbr,), device_id_type=pl.DeviceIdType.MESH,
                )
            pl.semaphore_wait(barrier_sem, 2)
            xcp.wait()

        @pl.when(j >= 1)
        def _finish_prev_front():
            for c in (0, 1):
                r2_old[c].wait()
            for c in (0, 1):
                q = P[c][3]
                out_ref[myq_rows(c), cols[c]] = (
                    har[c, pl.ds(q * Q, Q)] + s2r[pbm, c])
            for c in (0, 1):
                r3a[c].start()
                r3b[c].start()
                r4a[c].start()

        @pl.when(j < G)
        def _compute_and_ex1():
            wb[...] = w_ref[...].astype(jnp.bfloat16)
            for c in (0, 1):
                h = P[c][2]
                for sub in range(2):
                    rows = pl.ds((1 - h) * H + sub * Q, Q)
                    part[c, rows] = jnp.dot(
                        x_vmem[rows, :], wb[:, c * W:(c + 1) * W],
                        preferred_element_type=jnp.float32,
                    ).astype(jnp.bfloat16)
                r1[c].start()
            for c in (0, 1):
                h = P[c][2]
                for sub in range(2):
                    rows = pl.ds(h * H + sub * Q, Q)
                    part[c, rows] = jnp.dot(
                        x_vmem[rows, :], wb[:, c * W:(c + 1) * W],
                        preferred_element_type=jnp.float32,
                    ).astype(jnp.bfloat16)

        @pl.when(j < G)
        def _ex1_wait():
            for c in (0, 1):
                r1[c].wait()
            for c in (0, 1):
                h = P[c][2]
                har[c] = part[c, pl.ds(h * H, H)] + s1r[pb, c]

        @pl.when(j >= 1)
        def _finish_prev_tail():
            for c in (0, 1):
                r3a[c].wait()
            for c in (0, 1):
                r4ba[c].start()
            for c in (0, 1):
                r3b[c].wait()
            for c in (0, 1):
                r4bb[c].start()
            for c in (0, 1):
                r4a[c].wait()
                r4ba[c].wait()
                r4bb[c].wait()

        @pl.when(j < G)
        def _ex2_start():
            for c in (0, 1):
                r2_new[c].start()

    return pl.pallas_call(
        body,
        grid=(G + 1,),
        out_shape=jax.ShapeDtypeStruct((m, n), jnp.bfloat16),
        in_specs=[
            pl.BlockSpec(memory_space=pl.ANY),
            pl.BlockSpec((k_shard, NC), lambda j: (0, jnp.minimum(j, G - 1))),
        ],
        out_specs=pl.BlockSpec((m, NC), lambda j: (0, jnp.maximum(j - 1, 0))),
        scratch_shapes=[
            pltpu.VMEM((m, k_shard), jnp.bfloat16),
            pltpu.VMEM((k_shard, NC), jnp.bfloat16),
            pltpu.VMEM((2, m, W), jnp.bfloat16),
            pltpu.VMEM((2, 2, H, W), jnp.bfloat16),
            pltpu.VMEM((2, H, W), jnp.bfloat16),
            pltpu.VMEM((2, 2, Q, W), jnp.bfloat16),
            pltpu.SemaphoreType.DMA((2, 9)),
            pltpu.SemaphoreType.DMA((2, 9)),
            pltpu.SemaphoreType.DMA,
        ],
        compiler_params=pltpu.CompilerParams(
            collective_id=0,
            dimension_semantics=("arbitrary",),
            vmem_limit_bytes=100 * 1024 * 1024,
        ),
    )(x, w_mat)


# device time: 641599 ns/iter; 1.0187x vs baseline; 1.0187x over previous
import jax
import jax.numpy as jnp
from jax import lax
from jax.experimental import pallas as pl
from jax.experimental.pallas import tpu as pltpu

N_DEV = 4
NC = 1024


def kernel(x, w_mat):
    x = x.astype(jnp.bfloat16)
    m, k_shard = x.shape
    _, n = w_mat.shape
    G = n // NC
    W = NC // 2
    Wc = W // 2
    H = m // 2
    Q = m // 4

    def body(x_hbm, w_ref, out_ref, x_vmem, wb, part, s1r, har, s2r,
             ssem, rsem, xsem):
        my = lax.axis_index("i")
        j = pl.program_id(0)
        pb = j % 2
        pbm = 1 - pb

        def params(c):
            if c == 0:
                p1, p2 = my ^ 1, 3 - my
                h = (my & 1) ^ (my >> 1)
                q = my >> 1
            else:
                p1, p2 = 3 - my, my ^ 1
                h = my >> 1
                q = my & 1
            return p1, p2, h, q

        P = [params(0), params(1)]

        def copy(src, dst, c, k, dev):
            return pltpu.make_async_remote_copy(
                src_ref=src, dst_ref=dst,
                send_sem=ssem.at[c, k], recv_sem=rsem.at[c, k],
                device_id=(dev,), device_id_type=pl.DeviceIdType.MESH,
            )

        def myq_rows(c):
            _, _, h, q = P[c]
            return pl.ds((2 * h + q) * Q, Q)

        def otherq_rows(c):
            _, _, h, q = P[c]
            return pl.ds((2 * h + (1 - q)) * Q, Q)

        cols = [pl.ds(c * W, W) for c in (0, 1)]
        cols_a = [pl.ds(c * W, Wc) for c in (0, 1)]
        cols_b = [pl.ds(c * W + Wc, Wc) for c in (0, 1)]

        r1 = [copy(part.at[c, pl.ds((1 - P[c][2]) * H, H)],
                   s1r.at[pb, c], c, pb, P[c][0]) for c in (0, 1)]
        r2_new = [copy(har.at[c, pl.ds((1 - P[c][3]) * Q, Q)],
                       s2r.at[pb, c], c, 2 + pb, P[c][1]) for c in (0, 1)]
        r2_old = [copy(har.at[c, pl.ds((1 - P[c][3]) * Q, Q)],
                       s2r.at[pbm, c], c, 2 + pbm, P[c][1]) for c in (0, 1)]
        r3a = [copy(out_ref.at[myq_rows(c), cols_a[c]],
                    out_ref.at[myq_rows(c), cols_a[c]],
                    c, 4, P[c][1]) for c in (0, 1)]
        r3b = [copy(out_ref.at[myq_rows(c), cols_b[c]],
                    out_ref.at[myq_rows(c), cols_b[c]],
                    c, 5, P[c][1]) for c in (0, 1)]
        r4a = [copy(out_ref.at[myq_rows(c), cols[c]],
                    out_ref.at[myq_rows(c), cols[c]],
                    c, 6, P[c][0]) for c in (0, 1)]
        r4ba = [copy(out_ref.at[otherq_rows(c), cols_a[c]],
                     out_ref.at[otherq_rows(c), cols_a[c]],
                     c, 7, P[c][0]) for c in (0, 1)]
        r4bb = [copy(out_ref.at[otherq_rows(c), cols_b[c]],
                     out_ref.at[otherq_rows(c), cols_b[c]],
                     c, 8, P[c][0]) for c in (0, 1)]

        barrier_sem = pltpu.get_barrier_semaphore()

        @pl.when(j == 0)
        def _entry():
            xcp = pltpu.make_async_copy(x_hbm, x_vmem, xsem)
            xcp.start()
            for nbr in (my ^ 1, 3 - my):
                pl.semaphore_signal(
                    barrier_sem, inc=1,
                    device_id=(nbr,), device_id_type=pl.DeviceIdType.MESH,
                )
            pl.semaphore_wait(barrier_sem, 2)
            xcp.wait()

        @pl.when(j >= 1)
        def _finish_prev_front():
            for c in (0, 1):
                r2_old[c].wait()
            for c in (0, 1):
                q = P[c][3]
                out_ref[myq_rows(c), cols[c]] = (
                    har[c, pl.ds(q * Q, Q)] + s2r[pbm, c])
            for c in (0, 1):
                r3a[c].start()
                r3b[c].start()
                r4a[c].start()

        @pl.when(j < G)
        def _compute_and_ex1():
            wb[...] = w_ref[...].astype(jnp.bfloat16)
            for c in (0, 1):
                h = P[c][2]
                for sub in range(2):
                    rows = pl.ds((1 - h) * H + sub * Q, Q)
                    part[c, rows] = jnp.dot(
                        x_vmem[rows, :], wb[:, c * W:(c + 1) * W],
                        preferred_element_type=jnp.float32,
                    ).astype(jnp.bfloat16)
                r1[c].start()
            for c in (0, 1):
                h = P[c][2]
                for sub in range(2):
                    rows = pl.ds(h * H + sub * Q, Q)
                    part[c, rows] = jnp.dot(
                        x_vmem[rows, :], wb[:, c * W:(c + 1) * W],
                        preferred_element_type=jnp.float32,
                    ).astype(jnp.bfloat16)

        @pl.when(j < G)
        def _ex1_wait():
            for c in (0, 1):
                r1[c].wait()
            for c in (0, 1):
                h = P[c][2]
                har[c] = part[c, pl.ds(h * H, H)] + s1r[pb, c]

        @pl.when(j >= 1)
        def _finish_prev_relay():
            for c in (0, 1):
                r3a[c].wait()
            for c in (0, 1):
                r4ba[c].start()
            for c in (0, 1):
                r3b[c].wait()
            for c in (0, 1):
                r4bb[c].start()

        @pl.when(j < G)
        def _ex2_start():
            for c in (0, 1):
                r2_new[c].start()

        @pl.when(j >= 1)
        def _finish_prev_waits():
            for c in (0, 1):
                r4a[c].wait()
                r4ba[c].wait()
                r4bb[c].wait()

    return pl.pallas_call(
        body,
        grid=(G + 1,),
        out_shape=jax.ShapeDtypeStruct((m, n), jnp.bfloat16),
        in_specs=[
            pl.BlockSpec(memory_space=pl.ANY),
            pl.BlockSpec((k_shard, NC), lambda j: (0, jnp.minimum(j, G - 1))),
        ],
        out_specs=pl.BlockSpec((m, NC), lambda j: (0, jnp.maximum(j - 1, 0))),
        scratch_shapes=[
            pltpu.VMEM((m, k_shard), jnp.bfloat16),
            pltpu.VMEM((k_shard, NC), jnp.bfloat16),
            pltpu.VMEM((2, m, W), jnp.bfloat16),
            pltpu.VMEM((2, 2, H, W), jnp.bfloat16),
            pltpu.VMEM((2, H, W), jnp.bfloat16),
            pltpu.VMEM((2, 2, Q, W), jnp.bfloat16),
            pltpu.SemaphoreType.DMA((2, 9)),
            pltpu.SemaphoreType.DMA((2, 9)),
            pltpu.SemaphoreType.DMA,
        ],
        compiler_params=pltpu.CompilerParams(
            collective_id=0,
            dimension_semantics=("arbitrary",),
            vmem_limit_bytes=100 * 1024 * 1024,
        ),
    )(x, w_mat)


# device time: 613354 ns/iter; 1.0657x vs baseline; 1.0461x over previous
import jax
import jax.numpy as jnp
from jax import lax
from jax.experimental import pallas as pl
from jax.experimental.pallas import tpu as pltpu

N_DEV = 4
NC = 1024


def kernel(x, w_mat):
    x = x.astype(jnp.bfloat16)
    m, k_shard = x.shape
    _, n = w_mat.shape
    G = n // NC
    W = NC // 2
    Wc = W // 2
    H = m // 2
    Q = m // 4

    def body(x_hbm, w_ref, out_ref, x_vmem, wb, part, s1r, har, s2r,
             ssem, rsem, xsem):
        my = lax.axis_index("i")
        j = pl.program_id(0)
        pb = j % 2
        pbm = 1 - pb

        def params(c):
            if c == 0:
                p1, p2 = my ^ 1, 3 - my
                h = (my & 1) ^ (my >> 1)
                q = my >> 1
            else:
                p1, p2 = 3 - my, my ^ 1
                h = my >> 1
                q = my & 1
            return p1, p2, h, q

        P = [params(0), params(1)]

        def copy(src, dst, c, k, dev):
            return pltpu.make_async_remote_copy(
                src_ref=src, dst_ref=dst,
                send_sem=ssem.at[c, k], recv_sem=rsem.at[c, k],
                device_id=(dev,), device_id_type=pl.DeviceIdType.MESH,
            )

        def myq_rows(c):
            _, _, h, q = P[c]
            return pl.ds((2 * h + q) * Q, Q)

        def otherq_rows(c):
            _, _, h, q = P[c]
            return pl.ds((2 * h + (1 - q)) * Q, Q)

        cols = [pl.ds(c * W, W) for c in (0, 1)]
        cols_a = [pl.ds(c * W, Wc) for c in (0, 1)]
        cols_b = [pl.ds(c * W + Wc, Wc) for c in (0, 1)]

        r1 = [copy(part.at[c, pl.ds((1 - P[c][2]) * H, H)],
                   s1r.at[pb, c], c, pb, P[c][0]) for c in (0, 1)]
        r2_new = [copy(har.at[c, pl.ds((1 - P[c][3]) * Q, Q)],
                       s2r.at[pb, c], c, 2 + pb, P[c][1]) for c in (0, 1)]
        r2_old = [copy(har.at[c, pl.ds((1 - P[c][3]) * Q, Q)],
                       s2r.at[pbm, c], c, 2 + pbm, P[c][1]) for c in (0, 1)]
        r3a = [copy(out_ref.at[myq_rows(c), cols_a[c]],
                    out_ref.at[myq_rows(c), cols_a[c]],
                    c, 4, P[c][1]) for c in (0, 1)]
        r3b = [copy(out_ref.at[myq_rows(c), cols_b[c]],
                    out_ref.at[myq_rows(c), cols_b[c]],
                    c, 5, P[c][1]) for c in (0, 1)]
        r4a = [copy(out_ref.at[myq_rows(c), cols[c]],
                    out_ref.at[myq_rows(c), cols[c]],
                    c, 6, P[c][0]) for c in (0, 1)]
        r4ba = [copy(out_ref.at[otherq_rows(c), cols_a[c]],
                     out_ref.at[otherq_rows(c), cols_a[c]],
                     c, 7, P[c][0]) for c in (0, 1)]
        r4bb = [copy(out_ref.at[otherq_rows(c), cols_b[c]],
                     out_ref.at[otherq_rows(c), cols_b[c]],
                     c, 8, P[c][0]) for c in (0, 1)]

        barrier_sem = pltpu.get_barrier_semaphore()

        @pl.when(j == 0)
        def _entry():
            xcp = pltpu.make_async_copy(x_hbm, x_vmem, xsem)
            xcp.start()
            for nbr in (my ^ 1, 3 - my):
                pl.semaphore_signal(
                    barrier_sem, inc=1,
                    device_id=(nbr,), device_id_type=pl.DeviceIdType.MESH,
                )
            pl.semaphore_wait(barrier_sem, 2)
            xcp.wait()

        @pl.when(j < G)
        def _compute_and_ex1():
            wb[...] = w_ref[...].astype(jnp.bfloat16)
            for c in (0, 1):
                h = P[c][2]
                for sub in range(2):
                    rows = pl.ds((1 - h) * H + sub * Q, Q)
                    part[c, rows] = jnp.dot(
                        x_vmem[rows, :], wb[:, c * W:(c + 1) * W],
                        preferred_element_type=jnp.float32,
                    ).astype(jnp.bfloat16)
                r1[c].start()
            for c in (0, 1):
                h = P[c][2]
                for sub in range(2):
                    rows = pl.ds(h * H + sub * Q, Q)
                    part[c, rows] = jnp.dot(
                        x_vmem[rows, :], wb[:, c * W:(c + 1) * W],
                        preferred_element_type=jnp.float32,
                    ).astype(jnp.bfloat16)

        @pl.when(j >= 1)
        def _finish_prev_front():
            for c in (0, 1):
                r2_old[c].wait()
            for c in (0, 1):
                q = P[c][3]
                out_ref[myq_rows(c), cols[c]] = (
                    har[c, pl.ds(q * Q, Q)] + s2r[pbm, c])
            for c in (0, 1):
                r3a[c].start()
                r3b[c].start()
                r4a[c].start()

        @pl.when(j < G)
        def _ex1_wait():
            for c in (0, 1):
                r1[c].wait()
            for c in (0, 1):
                h = P[c][2]
                har[c] = part[c, pl.ds(h * H, H)] + s1r[pb, c]

        @pl.when(j >= 1)
        def _finish_prev_relay():
            for c in (0, 1):
                r3a[c].wait()
            for c in (0, 1):
                r4ba[c].start()
            for c in (0, 1):
                r3b[c].wait()
            for c in (0, 1):
                r4bb[c].start()

        @pl.when(j < G)
        def _ex2_start():
            for c in (0, 1):
                r2_new[c].start()

        @pl.when(j >= 1)
        def _finish_prev_waits():
            for c in (0, 1):
                r4a[c].wait()
                r4ba[c].wait()
                r4bb[c].wait()

    return pl.pallas_call(
        body,
        grid=(G + 1,),
        out_shape=jax.ShapeDtypeStruct((m, n), jnp.bfloat16),
        in_specs=[
            pl.BlockSpec(memory_space=pl.ANY),
            pl.BlockSpec((k_shard, NC), lambda j: (0, jnp.minimum(j, G - 1))),
        ],
        out_specs=pl.BlockSpec((m, NC), lambda j: (0, jnp.maximum(j - 1, 0))),
        scratch_shapes=[
            pltpu.VMEM((m, k_shard), jnp.bfloat16),
            pltpu.VMEM((k_shard, NC), jnp.bfloat16),
            pltpu.VMEM((2, m, W), jnp.bfloat16),
            pltpu.VMEM((2, 2, H, W), jnp.bfloat16),
            pltpu.VMEM((2, H, W), jnp.bfloat16),
            pltpu.VMEM((2, 2, Q, W), jnp.bfloat16),
            pltpu.SemaphoreType.DMA((2, 9)),
            pltpu.SemaphoreType.DMA((2, 9)),
            pltpu.SemaphoreType.DMA,
        ],
        compiler_params=pltpu.CompilerParams(
            collective_id=0,
            dimension_semantics=("arbitrary",),
            vmem_limit_bytes=100 * 1024 * 1024,
        ),
    )(x, w_mat)
